# baseline (speedup 1.0000x reference)
import functools

import jax
import jax.numpy as jnp
from jax import lax
from jax.experimental import pallas as pl
from jax.experimental.pallas import tpu as pltpu
from jax.experimental.pallas import tpu_sc as plsc

BATCH = 16384
EMBED_DIM = 16
HIDDEN_DIM = 64
GROUP = 8
GROUP_W = GROUP * EMBED_DIM
NGROUPS = 1000000 // GROUP
NC = 2
NS = 16
NW = NC * NS
B_PER_W = BATCH // NW
STREAM = 128
N_CHUNK = B_PER_W // STREAM


def _sc_gather(ut128, it128, uids, iids):
    mesh = plsc.VectorSubcoreMesh(core_axis_name="c", subcore_axis_name="s")

    @functools.partial(
        pl.kernel,
        mesh=mesh,
        out_type=[
            jax.ShapeDtypeStruct((BATCH, GROUP_W), jnp.float32),
            jax.ShapeDtypeStruct((BATCH, GROUP_W), jnp.float32),
        ],
        scratch_types=[
            pltpu.VMEM((N_CHUNK, STREAM), jnp.int32),
            pltpu.VMEM((N_CHUNK, STREAM), jnp.int32),
            pltpu.VMEM((STREAM, GROUP_W), jnp.float32),
            pltpu.VMEM((STREAM, GROUP_W), jnp.float32),
            pltpu.SemaphoreType.DMA,
            pltpu.SemaphoreType.DMA,
        ],
    )
    def k(utab_hbm, itab_hbm, uid_hbm, iid_hbm, uout_hbm, iout_hbm,
          uidx_v, iidx_v, urows_v, irows_v, usem, isem):
        wid = lax.axis_index("s") * NC + lax.axis_index("c")
        base = wid * B_PER_W
        pltpu.sync_copy(uid_hbm.at[wid], uidx_v)
        pltpu.sync_copy(iid_hbm.at[wid], iidx_v)
        for c in range(N_CHUNK):
            ucp = pltpu.async_copy(utab_hbm.at[uidx_v.at[c]], urows_v, usem)
            icp = pltpu.async_copy(itab_hbm.at[iidx_v.at[c]], irows_v, isem)
            ucp.wait()
            icp.wait()
            pltpu.sync_copy(
                urows_v, uout_hbm.at[pl.ds(base + c * STREAM, STREAM)])
            pltpu.sync_copy(
                irows_v, iout_hbm.at[pl.ds(base + c * STREAM, STREAM)])

    return k(ut128, it128, uids, iids)


def _mlp_body(u_ref, i_ref, ur_ref, ir_ref, w1u_ref, w1i_ref, b1_ref,
              w2_ref, b2_ref, w3_ref, b3_ref, out_ref):
    seg = jax.lax.broadcasted_iota(
        jnp.int32, (1, GROUP_W), 1) // EMBED_DIM
    um = jnp.where(seg == ur_ref[...], u_ref[...], 0.0)
    im = jnp.where(seg == ir_ref[...], i_ref[...], 0.0)
    h1 = jnp.dot(um, w1u_ref[...], preferred_element_type=jnp.float32)
    h1 += jnp.dot(im, w1i_ref[...], preferred_element_type=jnp.float32)
    h1 = jax.nn.relu(h1 + b1_ref[...])
    h2 = jax.nn.relu(
        jnp.dot(h1, w2_ref[...], preferred_element_type=jnp.float32)
        + b2_ref[...])
    o = jnp.sum(h2 * w3_ref[...], axis=1, keepdims=True) + b3_ref[...]
    out_ref[...] = jax.nn.sigmoid(o)


def _tc_mlp(u128, i128, ur, ir, W1, b1, W2, b2, W3, b3):
    blk = 2048
    grid = (BATCH // blk,)
    w1u8 = jnp.tile(W1[:, :EMBED_DIM].T, (GROUP, 1))
    w1i8 = jnp.tile(W1[:, EMBED_DIM:].T, (GROUP, 1))
    w2 = W2.T
    b1r = b1.reshape(1, HIDDEN_DIM)
    b2r = b2.reshape(1, HIDDEN_DIM)
    w3r = W3.reshape(1, HIDDEN_DIM)
    b3r = b3.reshape(1, 1)
    full = lambda shape: pl.BlockSpec(shape, lambda i: (0, 0))
    return pl.pallas_call(
        _mlp_body,
        grid=grid,
        in_specs=[
            pl.BlockSpec((blk, GROUP_W), lambda i: (i, 0)),
            pl.BlockSpec((blk, GROUP_W), lambda i: (i, 0)),
            pl.BlockSpec((blk, 1), lambda i: (i, 0)),
            pl.BlockSpec((blk, 1), lambda i: (i, 0)),
            full((GROUP_W, HIDDEN_DIM)),
            full((GROUP_W, HIDDEN_DIM)),
            full((1, HIDDEN_DIM)),
            full((HIDDEN_DIM, HIDDEN_DIM)),
            full((1, HIDDEN_DIM)),
            full((1, HIDDEN_DIM)),
            full((1, 1)),
        ],
        out_specs=pl.BlockSpec((blk, 1), lambda i: (i, 0)),
        out_shape=jax.ShapeDtypeStruct((BATCH, 1), jnp.float32),
    )(u128, i128, ur, ir, w1u8, w1i8, b1r, w2, b2r, w3r, b3r)


def kernel(user_ids, item_ids, user_table, item_table, W1, b1, W2, b2, W3, b3):
    uid = user_ids.astype(jnp.int32)
    iid = item_ids.astype(jnp.int32)
    ut128 = user_table.reshape(NGROUPS, GROUP_W)
    it128 = item_table.reshape(NGROUPS, GROUP_W)
    ugrp = (uid // GROUP).reshape(NW, N_CHUNK, STREAM)
    igrp = (iid // GROUP).reshape(NW, N_CHUNK, STREAM)
    ur = (uid % GROUP).reshape(BATCH, 1)
    ir = (iid % GROUP).reshape(BATCH, 1)
    u128, i128 = _sc_gather(ut128, it128, ugrp, igrp)
    return _tc_mlp(u128, i128, ur, ir, W1, b1, W2, b2, W3, b3)


# speedup vs baseline: 1.0147x; 1.0147x over previous
import functools

import jax
import jax.numpy as jnp
from jax import lax
from jax.experimental import pallas as pl
from jax.experimental.pallas import tpu as pltpu
from jax.experimental.pallas import tpu_sc as plsc

BATCH = 16384
EMBED_DIM = 16
HIDDEN_DIM = 64
NC = 2
NS = 16
NW = NC * NS
B_PER_W = BATCH // NW
STREAM = 128
N_CHUNK = B_PER_W // STREAM


def _sc_gather_one(table, ids):
    mesh = plsc.VectorSubcoreMesh(core_axis_name="c", subcore_axis_name="s")

    @functools.partial(
        pl.kernel,
        mesh=mesh,
        compiler_params=pltpu.CompilerParams(use_tc_tiling_on_sc=False),
        out_type=jax.ShapeDtypeStruct((BATCH, EMBED_DIM), jnp.float32),
        scratch_types=[
            pltpu.VMEM((N_CHUNK, STREAM), jnp.int32),
            pltpu.VMEM((B_PER_W, EMBED_DIM), jnp.float32),
            pltpu.SemaphoreType.DMA,
        ],
    )
    def k(tab_hbm, id_hbm, out_hbm, idx_v, rows_v, sem):
        wid = lax.axis_index("s") * NC + lax.axis_index("c")
        base = wid * B_PER_W
        pltpu.sync_copy(id_hbm.at[wid], idx_v)
        copies = []
        for c in range(N_CHUNK):
            copies.append(pltpu.async_copy(
                tab_hbm.at[idx_v.at[c]],
                rows_v.at[pl.ds(c * STREAM, STREAM)], sem))
        for cp in copies:
            cp.wait()
        pltpu.sync_copy(rows_v, out_hbm.at[pl.ds(base, B_PER_W)])

    return k(table, ids)


def _mlp_body(ue_ref, ie_ref, w1u_ref, w1i_ref, b1_ref, w2_ref, b2_ref,
              w3_ref, b3_ref, out_ref):
    h1 = jnp.dot(ue_ref[...], w1u_ref[...], preferred_element_type=jnp.float32)
    h1 += jnp.dot(ie_ref[...], w1i_ref[...], preferred_element_type=jnp.float32)
    h1 = jax.nn.relu(h1 + b1_ref[...])
    h2 = jax.nn.relu(
        jnp.dot(h1, w2_ref[...], preferred_element_type=jnp.float32)
        + b2_ref[...])
    o = jnp.sum(h2 * w3_ref[...], axis=1, keepdims=True) + b3_ref[...]
    out_ref[...] = jax.nn.sigmoid(o)


def _tc_mlp(ue, ie, W1, b1, W2, b2, W3, b3):
    blk = 2048
    grid = (BATCH // blk,)
    w1u = W1[:, :EMBED_DIM].T
    w1i = W1[:, EMBED_DIM:].T
    w2 = W2.T
    b1r = b1.reshape(1, HIDDEN_DIM)
    b2r = b2.reshape(1, HIDDEN_DIM)
    w3r = W3.reshape(1, HIDDEN_DIM)
    b3r = b3.reshape(1, 1)
    full = lambda shape: pl.BlockSpec(shape, lambda i: (0, 0))
    return pl.pallas_call(
        _mlp_body,
        grid=grid,
        in_specs=[
            pl.BlockSpec((blk, EMBED_DIM), lambda i: (i, 0)),
            pl.BlockSpec((blk, EMBED_DIM), lambda i: (i, 0)),
            full((EMBED_DIM, HIDDEN_DIM)),
            full((EMBED_DIM, HIDDEN_DIM)),
            full((1, HIDDEN_DIM)),
            full((HIDDEN_DIM, HIDDEN_DIM)),
            full((1, HIDDEN_DIM)),
            full((1, HIDDEN_DIM)),
            full((1, 1)),
        ],
        out_specs=pl.BlockSpec((blk, 1), lambda i: (i, 0)),
        out_shape=jax.ShapeDtypeStruct((BATCH, 1), jnp.float32),
    )(ue, ie, w1u, w1i, b1r, w2, b2r, w3r, b3r)


def kernel(user_ids, item_ids, user_table, item_table, W1, b1, W2, b2, W3, b3):
    uids = user_ids.astype(jnp.int32).reshape(NW, N_CHUNK, STREAM)
    iids = item_ids.astype(jnp.int32).reshape(NW, N_CHUNK, STREAM)
    ue = _sc_gather_one(user_table, uids)
    ie = _sc_gather_one(item_table, iids)
    return _tc_mlp(ue, ie, W1, b1, W2, b2, W3, b3)
